# baseline (speedup 1.0000x reference)
import functools

import jax
import jax.numpy as jnp
from jax.experimental import pallas as pl
from jax.experimental.pallas import tpu as pltpu

B, N, D_IN, D_HID = 512, 128, 768, 16
G = 2


def _body(head_ref, x_ref, w1_ref, b1_ref, w2_ref, out_ref):
    x = x_ref[0]
    hd = head_ref[0]

    goff = jax.lax.broadcasted_iota(jnp.int32, (G, N), 0) * N
    dst = (hd + goff).reshape(1, G * N)
    row = jax.lax.broadcasted_iota(jnp.int32, (G * N, G * N), 0)
    st = jnp.where(row == dst, 1.0, 0.0)

    deg = 1.0 + jnp.sum(st, axis=1, keepdims=True)
    dinv = jax.lax.rsqrt(deg)

    def agg(v):
        u = v * dinv
        return dinv * (jnp.dot(st, u, preferred_element_type=jnp.float32) + u)

    h = jnp.dot(x, w1_ref[...], preferred_element_type=jnp.float32)
    h1 = jnp.maximum(agg(h) + b1_ref[...], 0.0)
    a2 = agg(h1)
    m = jnp.dot(a2, w2_ref[...], preferred_element_type=jnp.float32)

    m3 = m.reshape(G, N, D_IN)
    mx = jnp.max(m3, axis=1, keepdims=True)
    lse = mx + jnp.log(jnp.sum(jnp.exp(m3 - mx), axis=1, keepdims=True))
    out_ref[0] = (m3 - lse).reshape(G * N, D_IN)


@jax.jit
def kernel(head, x, W1, b1, W2, b2):
    del b2
    xf = x.reshape(B * N, D_IN)
    hd3 = head.reshape(B // G, G, N)
    out = pl.pallas_call(
        _body,
        grid=(B // G,),
        in_specs=[
            pl.BlockSpec((1, G, N), lambda i: (i, 0, 0)),
            pl.BlockSpec((1, G * N, D_IN), lambda i: (i, 0, 0)),
            pl.BlockSpec((D_IN, D_HID), lambda i: (0, 0)),
            pl.BlockSpec((1, D_HID), lambda i: (0, 0)),
            pl.BlockSpec((D_HID, D_IN), lambda i: (0, 0)),
        ],
        out_specs=pl.BlockSpec((1, G * N, D_IN), lambda i: (i, 0, 0)),
        out_shape=jax.ShapeDtypeStruct((B // G, G * N, D_IN), jnp.float32),
        compiler_params=pltpu.CompilerParams(
            dimension_semantics=("arbitrary",),
        ),
    )(hd3, xf.reshape(B // G, G * N, D_IN), W1, b1.reshape(1, D_HID), W2)
    return out.reshape(B, N, D_IN)


# speedup vs baseline: 1.3878x; 1.3878x over previous
import functools

import jax
import jax.numpy as jnp
from jax.experimental import pallas as pl
from jax.experimental.pallas import tpu as pltpu

B, N, D_IN, D_HID = 512, 128, 768, 16
G = 4


def _body(head_ref, x_ref, w1_ref, b1_ref, w2_ref, out_ref):
    x = x_ref[0]
    hd = head_ref[0]

    goff = jax.lax.broadcasted_iota(jnp.int32, (G, N), 0) * N
    dst = (hd + goff).reshape(1, G * N)
    row = jax.lax.broadcasted_iota(jnp.int32, (G * N, G * N), 0)
    st = jnp.where(row == dst, 1.0, 0.0)

    deg = 1.0 + jnp.sum(st, axis=1, keepdims=True)
    dinv = jax.lax.rsqrt(deg)

    def agg(v):
        u = v * dinv
        return dinv * (jnp.dot(st, u, preferred_element_type=jnp.float32) + u)

    h = jnp.dot(x, w1_ref[...], preferred_element_type=jnp.float32)
    h1 = jnp.maximum(agg(h) + b1_ref[...], 0.0)
    a2 = agg(h1)
    m = jnp.dot(a2, w2_ref[...], preferred_element_type=jnp.float32)

    m3 = m.reshape(G, N, D_IN)
    mx = jnp.max(m3, axis=1, keepdims=True)
    lse = mx + jnp.log(jnp.sum(jnp.exp(m3 - mx), axis=1, keepdims=True))
    out_ref[0] = (m3 - lse).reshape(G * N, D_IN)


@jax.jit
def kernel(head, x, W1, b1, W2, b2):
    del b2
    xf = x.reshape(B * N, D_IN)
    hd3 = head.reshape(B // G, G, N)
    out = pl.pallas_call(
        _body,
        grid=(B // G,),
        in_specs=[
            pl.BlockSpec((1, G, N), lambda i: (i, 0, 0)),
            pl.BlockSpec((1, G * N, D_IN), lambda i: (i, 0, 0)),
            pl.BlockSpec((D_IN, D_HID), lambda i: (0, 0)),
            pl.BlockSpec((1, D_HID), lambda i: (0, 0)),
            pl.BlockSpec((D_HID, D_IN), lambda i: (0, 0)),
        ],
        out_specs=pl.BlockSpec((1, G * N, D_IN), lambda i: (i, 0, 0)),
        out_shape=jax.ShapeDtypeStruct((B // G, G * N, D_IN), jnp.float32),
        compiler_params=pltpu.CompilerParams(
            dimension_semantics=("parallel",),
        ),
    )(hd3, xf.reshape(B // G, G * N, D_IN), W1, b1.reshape(1, D_HID), W2)
    return out.reshape(B, N, D_IN)


# speedup vs baseline: 1.4881x; 1.0723x over previous
import functools

import jax
import jax.numpy as jnp
from jax.experimental import pallas as pl
from jax.experimental.pallas import tpu as pltpu

B, N, D_IN, D_HID = 512, 128, 768, 16
G = 4


P = 2 * N


def _body(head_ref, x_ref, w1_ref, b1_ref, w2_ref, out_ref):
    x = x_ref[0]
    hd = head_ref[0]
    GN = G * N

    goff = jax.lax.broadcasted_iota(jnp.int32, (G, N), 0) * N
    dst = (hd + goff).reshape(1, GN)
    sts = []
    for k in range(GN // P):
        d = jax.lax.slice(dst, (0, k * P), (1, (k + 1) * P)) - k * P
        row = jax.lax.broadcasted_iota(jnp.int32, (P, P), 0)
        sts.append(
            jnp.where(row == d, 1.0, 0.0).astype(jnp.bfloat16)
        )

    cnt = jnp.concatenate(
        [jnp.sum(st, axis=1, keepdims=True, dtype=jnp.float32) for st in sts],
        axis=0)
    deg = 1.0 + cnt
    dinv = jax.lax.rsqrt(deg)

    def agg(v):
        u = v * dinv
        ub = u.astype(jnp.bfloat16)
        parts = [
            jnp.dot(sts[k], jax.lax.slice(ub, (k * P, 0), ((k + 1) * P, D_HID)),
                    preferred_element_type=jnp.float32)
            for k in range(GN // P)
        ]
        return dinv * (jnp.concatenate(parts, axis=0) + u)

    w1b = w1_ref[...].astype(jnp.bfloat16)
    w2b = w2_ref[...].astype(jnp.bfloat16)
    h = jnp.dot(x.astype(jnp.bfloat16), w1b,
                preferred_element_type=jnp.float32)
    h1 = jnp.maximum(agg(h) + b1_ref[...], 0.0)
    a2 = agg(h1)
    m = jnp.dot(a2.astype(jnp.bfloat16), w2b,
                preferred_element_type=jnp.float32)

    m3 = m.reshape(G, N, D_IN)
    mx = jnp.max(m3, axis=1, keepdims=True)
    lse = mx + jnp.log(jnp.sum(jnp.exp(m3 - mx), axis=1, keepdims=True))
    out_ref[0] = (m3 - lse).reshape(G * N, D_IN)


@jax.jit
def kernel(head, x, W1, b1, W2, b2):
    del b2
    xf = x.reshape(B * N, D_IN)
    hd3 = head.reshape(B // G, G, N)
    out = pl.pallas_call(
        _body,
        grid=(B // G,),
        in_specs=[
            pl.BlockSpec((1, G, N), lambda i: (i, 0, 0)),
            pl.BlockSpec((1, G * N, D_IN), lambda i: (i, 0, 0)),
            pl.BlockSpec((D_IN, D_HID), lambda i: (0, 0)),
            pl.BlockSpec((1, D_HID), lambda i: (0, 0)),
            pl.BlockSpec((D_HID, D_IN), lambda i: (0, 0)),
        ],
        out_specs=pl.BlockSpec((1, G * N, D_IN), lambda i: (i, 0, 0)),
        out_shape=jax.ShapeDtypeStruct((B // G, G * N, D_IN), jnp.float32),
        compiler_params=pltpu.CompilerParams(
            dimension_semantics=("parallel",),
        ),
    )(hd3, xf.reshape(B // G, G * N, D_IN), W1, b1.reshape(1, D_HID), W2)
    return out.reshape(B, N, D_IN)
